# baseline (speedup 1.0000x reference)
import functools

import jax
import jax.numpy as jnp
from jax import lax
from jax.experimental import pallas as pl
from jax.experimental.pallas import tpu as pltpu
from jax.experimental.pallas import tpu_sc as plsc

D = 300
B = 1024
CTX = 20
NROWS = B * CTX


@functools.lru_cache(maxsize=None)
def _make_sc_gather(n_rows: int, d: int):
    info = plsc.get_sparse_core_info()
    nw = info.num_cores * info.num_subcores
    rows_per_w = n_rows // nw
    chunk = 128
    n_chunks = rows_per_w // chunk
    assert rows_per_w % chunk == 0

    mesh = plsc.VectorSubcoreMesh(core_axis_name="c", subcore_axis_name="s")

    @functools.partial(
        pl.kernel,
        mesh=mesh,
        compiler_params=pltpu.CompilerParams(use_tc_tiling_on_sc=False),
        out_type=jax.ShapeDtypeStruct((n_rows, d), jnp.float32),
        scratch_types=[
            pltpu.VMEM((chunk,), jnp.int32),
            pltpu.VMEM((chunk, d), jnp.float32),
            pltpu.SemaphoreType.DMA,
        ],
    )
    def sc_gather(table_hbm, idx_hbm, out_hbm, idx_v, rows_v, sem):
        wid = lax.axis_index("s") * info.num_cores + lax.axis_index("c")
        base = wid * rows_per_w
        for c in range(n_chunks):
            off = base + c * chunk
            pltpu.sync_copy(idx_hbm.at[pl.ds(off, chunk)], idx_v)
            pltpu.async_copy(table_hbm.at[idx_v], rows_v, sem).wait()
            pltpu.sync_copy(rows_v, out_hbm.at[pl.ds(off, chunk)])

    return sc_gather


def _pool_body(g_ref, x_ref):
    g = g_ref[...]
    ss = jnp.sum(g * g, axis=-1, keepdims=True)
    norm = jnp.sqrt(ss)
    scale = jnp.minimum(1.0, 1.0 / jnp.maximum(norm, 1e-12))
    x_ref[...] = jnp.mean(g * scale, axis=1)


def _pool(gathered):
    bt = 256
    return pl.pallas_call(
        _pool_body,
        grid=(B // bt,),
        in_specs=[pl.BlockSpec((bt, CTX, D), lambda i: (i, 0, 0))],
        out_specs=pl.BlockSpec((bt, D), lambda i: (i, 0)),
        out_shape=jax.ShapeDtypeStruct((B, D), jnp.float32),
    )(gathered)


def _matmul_body(x_ref, w_ref, b_ref, o_ref):
    acc = lax.dot_general(
        x_ref[...], w_ref[...], (((1,), (1,)), ((), ())),
        preferred_element_type=jnp.float32,
    )
    o_ref[...] = acc + b_ref[...]


def _matmul(x, w, b2):
    v = w.shape[0]
    vt = 2048
    return pl.pallas_call(
        _matmul_body,
        grid=(pl.cdiv(v, vt),),
        in_specs=[
            pl.BlockSpec((B, D), lambda j: (0, 0)),
            pl.BlockSpec((vt, D), lambda j: (j, 0)),
            pl.BlockSpec((1, vt), lambda j: (0, j)),
        ],
        out_specs=pl.BlockSpec((B, vt), lambda j: (0, j)),
        out_shape=jax.ShapeDtypeStruct((B, v), jnp.float32),
    )(x, w, b2)


def kernel(inputs_, emb_table, W, b):
    idx_flat = inputs_.reshape(-1)
    gathered = _make_sc_gather(NROWS, D)(emb_table, idx_flat)
    x = _pool(gathered.reshape(B, CTX, D))
    return _matmul(x, W, b.reshape(1, -1))


# speedup vs baseline: 1.7674x; 1.7674x over previous
import functools

import jax
import jax.numpy as jnp
from jax import lax
from jax.experimental import pallas as pl
from jax.experimental.pallas import tpu as pltpu
from jax.experimental.pallas import tpu_sc as plsc

D = 300
B = 1024
CTX = 20
NROWS = B * CTX


@functools.lru_cache(maxsize=None)
def _make_sc_gather(n_rows: int, d: int):
    info = plsc.get_sparse_core_info()
    nw = info.num_cores * info.num_subcores
    rows_per_w = n_rows // nw
    ch = 16
    n_chunks = rows_per_w // ch
    assert rows_per_w % ch == 0

    mesh = plsc.VectorSubcoreMesh(core_axis_name="c", subcore_axis_name="s")

    @functools.partial(
        pl.kernel,
        mesh=mesh,
        out_type=jax.ShapeDtypeStruct((n_rows, d), jnp.float32),
        scratch_types=[
            pltpu.VMEM((rows_per_w,), jnp.int32),
            pltpu.VMEM((ch, d), jnp.float32),
            pltpu.SemaphoreType.DMA,
        ],
    )
    def sc_gather(table_hbm, idx_hbm, out_hbm, idx_v, rows_v, sem):
        wid = lax.axis_index("s") * info.num_cores + lax.axis_index("c")
        base = wid * rows_per_w
        pltpu.sync_copy(idx_hbm.at[pl.ds(base, rows_per_w)], idx_v)

        def chunk_body(c, carry):
            coff = c * ch
            idx_vec = idx_v[pl.ds(coff, ch)]
            descs = []
            for j in range(ch):
                r = idx_vec[j]
                descs.append(
                    pltpu.async_copy(
                        table_hbm.at[pl.ds(r, 1)], rows_v.at[pl.ds(j, 1)], sem
                    )
                )
            for dsc in descs:
                dsc.wait()
            pltpu.sync_copy(rows_v, out_hbm.at[pl.ds(base + coff, ch)])
            return carry

        lax.fori_loop(0, n_chunks, chunk_body, 0)

    return sc_gather


def _pool_body(g_ref, x_ref):
    g = g_ref[...]
    ss = jnp.sum(g * g, axis=-1, keepdims=True)
    norm = jnp.sqrt(ss)
    scale = jnp.minimum(1.0, 1.0 / jnp.maximum(norm, 1e-12))
    x_ref[...] = jnp.mean(g * scale, axis=1)


def _pool(gathered):
    bt = 256
    return pl.pallas_call(
        _pool_body,
        grid=(B // bt,),
        in_specs=[pl.BlockSpec((bt, CTX, D), lambda i: (i, 0, 0))],
        out_specs=pl.BlockSpec((bt, D), lambda i: (i, 0)),
        out_shape=jax.ShapeDtypeStruct((B, D), jnp.float32),
    )(gathered)


def _matmul_body(x_ref, w_ref, b_ref, o_ref):
    acc = lax.dot_general(
        x_ref[...], w_ref[...], (((1,), (1,)), ((), ())),
        preferred_element_type=jnp.float32,
    )
    o_ref[...] = acc + b_ref[...]


def _matmul(x, w, b2):
    v = w.shape[0]
    vt = 2048
    return pl.pallas_call(
        _matmul_body,
        grid=(pl.cdiv(v, vt),),
        in_specs=[
            pl.BlockSpec((B, D), lambda j: (0, 0)),
            pl.BlockSpec((vt, D), lambda j: (j, 0)),
            pl.BlockSpec((1, vt), lambda j: (0, j)),
        ],
        out_specs=pl.BlockSpec((B, vt), lambda j: (0, j)),
        out_shape=jax.ShapeDtypeStruct((B, v), jnp.float32),
    )(x, w, b2)


def kernel(inputs_, emb_table, W, b):
    idx_flat = inputs_.reshape(-1)
    gathered = _make_sc_gather(NROWS, D)(emb_table, idx_flat)
    x = _pool(gathered.reshape(B, CTX, D))
    return _matmul(x, W, b.reshape(1, -1))


# speedup vs baseline: 1.7732x; 1.0033x over previous
import functools

import jax
import jax.numpy as jnp
from jax import lax
from jax.experimental import pallas as pl
from jax.experimental.pallas import tpu as pltpu
from jax.experimental.pallas import tpu_sc as plsc

D = 300
B = 1024
CTX = 20
NROWS = B * CTX


@functools.lru_cache(maxsize=None)
def _make_sc_gather(n_rows: int, d: int):
    info = plsc.get_sparse_core_info()
    nw = info.num_cores * info.num_subcores
    rows_per_w = n_rows // nw
    ch = 16
    n_chunks = rows_per_w // ch
    assert rows_per_w % ch == 0

    mesh = plsc.VectorSubcoreMesh(core_axis_name="c", subcore_axis_name="s")

    @functools.partial(
        pl.kernel,
        mesh=mesh,
        out_type=jax.ShapeDtypeStruct((n_rows, d), jnp.float32),
        scratch_types=[
            pltpu.VMEM((rows_per_w,), jnp.int32),
            pltpu.VMEM((ch, d), jnp.float32),
            pltpu.SemaphoreType.DMA,
        ],
    )
    def sc_gather(table_hbm, idx_hbm, out_hbm, idx_v, rows_v, sem):
        wid = lax.axis_index("s") * info.num_cores + lax.axis_index("c")
        base = wid * rows_per_w
        pltpu.sync_copy(idx_hbm.at[pl.ds(base, rows_per_w)], idx_v)

        def chunk_body(c, carry):
            coff = c * ch
            idx_vec = idx_v[pl.ds(coff, ch)]
            descs = []
            for j in range(ch):
                r = idx_vec[j]
                descs.append(
                    pltpu.async_copy(
                        table_hbm.at[pl.ds(r, 1)], rows_v.at[pl.ds(j, 1)], sem
                    )
                )
            for dsc in descs:
                dsc.wait()
            pltpu.sync_copy(rows_v, out_hbm.at[pl.ds(base + coff, ch)])
            return carry

        lax.fori_loop(0, n_chunks, chunk_body, 0)

    return sc_gather


def _pool_body(g_ref, x_ref):
    g = g_ref[...]
    ss = jnp.sum(g * g, axis=-1, keepdims=True)
    norm = jnp.sqrt(ss)
    scale = jnp.minimum(1.0, 1.0 / jnp.maximum(norm, 1e-12))
    x_ref[...] = jnp.mean(g * scale, axis=1)


def _pool(gathered):
    bt = 256
    return pl.pallas_call(
        _pool_body,
        grid=(B // bt,),
        in_specs=[pl.BlockSpec((bt, CTX, D), lambda i: (i, 0, 0))],
        out_specs=pl.BlockSpec((bt, D), lambda i: (i, 0)),
        out_shape=jax.ShapeDtypeStruct((B, D), jnp.float32),
    )(gathered)


def _matmul_body(x_ref, w_ref, b_ref, o_ref):
    xb = x_ref[...].astype(jnp.bfloat16)
    wb = w_ref[...].astype(jnp.bfloat16)
    acc = lax.dot_general(
        xb, wb, (((1,), (1,)), ((), ())),
        preferred_element_type=jnp.float32,
    )
    o_ref[...] = acc + b_ref[...]


def _matmul(x, w, b2):
    v = w.shape[0]
    vt = 2048
    return pl.pallas_call(
        _matmul_body,
        grid=(pl.cdiv(v, vt),),
        in_specs=[
            pl.BlockSpec((B, D), lambda j: (0, 0)),
            pl.BlockSpec((vt, D), lambda j: (j, 0)),
            pl.BlockSpec((1, vt), lambda j: (0, j)),
        ],
        out_specs=pl.BlockSpec((B, vt), lambda j: (0, j)),
        out_shape=jax.ShapeDtypeStruct((B, v), jnp.float32),
    )(x, w, b2)


def kernel(inputs_, emb_table, W, b):
    idx_flat = inputs_.reshape(-1)
    gathered = _make_sc_gather(NROWS, D)(emb_table, idx_flat)
    x = _pool(gathered.reshape(B, CTX, D))
    return _matmul(x, W, b.reshape(1, -1))


# speedup vs baseline: 1.7798x; 1.0037x over previous
import functools

import jax
import jax.numpy as jnp
from jax import lax
from jax.experimental import pallas as pl
from jax.experimental.pallas import tpu as pltpu
from jax.experimental.pallas import tpu_sc as plsc

D = 300
B = 1024
CTX = 20
NROWS = B * CTX


@functools.lru_cache(maxsize=None)
def _make_sc_gather(n_rows: int, d: int):
    info = plsc.get_sparse_core_info()
    nw = info.num_cores * info.num_subcores
    rows_per_w = n_rows // nw
    ch = 16
    n_chunks = rows_per_w // ch
    assert rows_per_w % ch == 0

    mesh = plsc.VectorSubcoreMesh(core_axis_name="c", subcore_axis_name="s")

    @functools.partial(
        pl.kernel,
        mesh=mesh,
        out_type=jax.ShapeDtypeStruct((n_rows, d), jnp.float32),
        scratch_types=[
            pltpu.VMEM((rows_per_w,), jnp.int32),
            pltpu.VMEM((ch, d), jnp.float32),
            pltpu.SemaphoreType.DMA,
        ],
    )
    def sc_gather(table_hbm, idx_hbm, out_hbm, idx_v, rows_v, sem):
        wid = lax.axis_index("s") * info.num_cores + lax.axis_index("c")
        base = wid * rows_per_w
        pltpu.sync_copy(idx_hbm.at[pl.ds(base, rows_per_w)], idx_v)

        def chunk_body(c, carry):
            coff = c * ch
            idx_vec = idx_v[pl.ds(coff, ch)]
            descs = []
            for j in range(ch):
                r = idx_vec[j]
                descs.append(
                    pltpu.async_copy(
                        table_hbm.at[pl.ds(r, 1)], rows_v.at[pl.ds(j, 1)], sem
                    )
                )
            for dsc in descs:
                dsc.wait()
            pltpu.sync_copy(rows_v, out_hbm.at[pl.ds(base + coff, ch)])
            return carry

        lax.fori_loop(0, n_chunks, chunk_body, 0)

    return sc_gather


def _pool_body(g_ref, x_ref):
    g = g_ref[...]
    ss = jnp.sum(g * g, axis=-1, keepdims=True)
    norm = jnp.sqrt(ss)
    scale = jnp.minimum(1.0, 1.0 / jnp.maximum(norm, 1e-12))
    x_ref[...] = jnp.mean(g * scale, axis=1)


def _pool(gathered):
    bt = 256
    return pl.pallas_call(
        _pool_body,
        grid=(B // bt,),
        in_specs=[pl.BlockSpec((bt, CTX, D), lambda i: (i, 0, 0))],
        out_specs=pl.BlockSpec((bt, D), lambda i: (i, 0)),
        out_shape=jax.ShapeDtypeStruct((B, D), jnp.float32),
    )(gathered)


def _matmul_body(x_ref, w_ref, b_ref, o_ref):
    xb = x_ref[...].astype(jnp.bfloat16)
    wb = w_ref[...].astype(jnp.bfloat16)
    acc = lax.dot_general(
        xb, wb, (((1,), (1,)), ((), ())),
        preferred_element_type=jnp.float32,
    )
    o_ref[...] = acc + b_ref[...]


def _matmul(x, w, b2):
    v = w.shape[0]
    vt = 4096
    return pl.pallas_call(
        _matmul_body,
        grid=(pl.cdiv(v, vt),),
        in_specs=[
            pl.BlockSpec((B, D), lambda j: (0, 0)),
            pl.BlockSpec((vt, D), lambda j: (j, 0)),
            pl.BlockSpec((1, vt), lambda j: (0, j)),
        ],
        out_specs=pl.BlockSpec((B, vt), lambda j: (0, j)),
        out_shape=jax.ShapeDtypeStruct((B, v), jnp.float32),
    )(x, w, b2)


def kernel(inputs_, emb_table, W, b):
    idx_flat = inputs_.reshape(-1)
    gathered = _make_sc_gather(NROWS, D)(emb_table, idx_flat)
    x = _pool(gathered.reshape(B, CTX, D))
    return _matmul(x, W, b.reshape(1, -1))


# speedup vs baseline: 1.9041x; 1.0698x over previous
import functools

import jax
import jax.numpy as jnp
from jax import lax
from jax.experimental import pallas as pl
from jax.experimental.pallas import tpu as pltpu
from jax.experimental.pallas import tpu_sc as plsc

D = 300
B = 1024
CTX = 20
NROWS = B * CTX


_RSQRT_MAGIC = 0x5F3759DF


@functools.lru_cache(maxsize=None)
def _make_sc_pool():
    info = plsc.get_sparse_core_info()
    nw = info.num_cores * info.num_subcores
    bat_per_w = B // nw
    n_full = D // 16
    tail = D - n_full * 16
    tail_off = D - 16
    mesh = plsc.VectorSubcoreMesh(core_axis_name="c", subcore_axis_name="s")

    @functools.partial(
        pl.kernel,
        mesh=mesh,
        compiler_params=pltpu.CompilerParams(needs_layout_passes=False),
        out_type=jax.ShapeDtypeStruct((B * D,), jnp.float32),
        scratch_types=[
            pltpu.VMEM((bat_per_w * CTX,), jnp.int32),
            pltpu.VMEM((CTX, D), jnp.float32),
            pltpu.VMEM((bat_per_w * D,), jnp.float32),
            pltpu.SemaphoreType.DMA,
        ],
    )
    def sc_pool(table_hbm, idx_hbm, x_hbm, idx_v, rows_v, xst, sem):
        wid = lax.axis_index("s") * info.num_cores + lax.axis_index("c")
        pltpu.sync_copy(
            idx_hbm.at[pl.ds(wid * bat_per_w * CTX, bat_per_w * CTX)], idx_v
        )
        tail_mask = lax.iota(jnp.int32, 16) >= (16 - tail)

        def body(bl, carry):
            ioff = bl * CTX
            iv0 = idx_v[pl.ds(ioff, 16)]
            iv1 = idx_v[pl.ds(ioff + CTX - 16, 16)]
            descs = []
            for j in range(CTX):
                r = iv0[j] if j < 16 else iv1[j - (CTX - 16)]
                descs.append(
                    pltpu.async_copy(
                        table_hbm.at[pl.ds(r, 1)], rows_v.at[pl.ds(j, 1)], sem
                    )
                )
            for dsc in descs:
                dsc.wait()

            xacc = [jnp.zeros((16,), jnp.float32) for _ in range(n_full + 1)]
            for j in range(CTX):
                chunks = [rows_v[j, pl.ds(c * 16, 16)] for c in range(n_full)]
                last = rows_v[j, pl.ds(tail_off, 16)]
                last = jnp.where(tail_mask, last, 0.0)
                chunks.append(last)
                ssq = jnp.zeros((16,), jnp.float32)
                for ch_ in chunks:
                    ssq = ssq + ch_ * ch_
                s = jnp.sum(ssq)
                svec = jnp.full((16,), s, jnp.float32)
                ibits = lax.bitcast_convert_type(svec, jnp.int32)
                y = lax.bitcast_convert_type(
                    _RSQRT_MAGIC - lax.shift_right_logical(ibits, 1), jnp.float32
                )
                for _ in range(3):
                    y = y * (1.5 - 0.5 * svec * y * y)
                scale = jnp.where(svec > 1.0, y, 1.0)
                for c in range(n_full + 1):
                    xacc[c] = xacc[c] + chunks[c] * scale

            inv = jnp.float32(1.0 / CTX)
            xoff = bl * D
            xst[pl.ds(xoff + tail_off, 16)] = xacc[n_full] * inv
            for c in range(n_full):
                xst[pl.ds(xoff + c * 16, 16)] = xacc[c] * inv
            return carry

        lax.fori_loop(0, bat_per_w, body, 0)
        pltpu.sync_copy(xst, x_hbm.at[pl.ds(wid * bat_per_w * D, bat_per_w * D)])

    return sc_pool


@functools.lru_cache(maxsize=None)
def _make_sc_gather(n_rows: int, d: int):
    info = plsc.get_sparse_core_info()
    nw = info.num_cores * info.num_subcores
    rows_per_w = n_rows // nw
    ch = 16
    n_chunks = rows_per_w // ch
    assert rows_per_w % ch == 0

    mesh = plsc.VectorSubcoreMesh(core_axis_name="c", subcore_axis_name="s")

    @functools.partial(
        pl.kernel,
        mesh=mesh,
        out_type=jax.ShapeDtypeStruct((n_rows, d), jnp.float32),
        scratch_types=[
            pltpu.VMEM((rows_per_w,), jnp.int32),
            pltpu.VMEM((ch, d), jnp.float32),
            pltpu.SemaphoreType.DMA,
        ],
    )
    def sc_gather(table_hbm, idx_hbm, out_hbm, idx_v, rows_v, sem):
        wid = lax.axis_index("s") * info.num_cores + lax.axis_index("c")
        base = wid * rows_per_w
        pltpu.sync_copy(idx_hbm.at[pl.ds(base, rows_per_w)], idx_v)

        def chunk_body(c, carry):
            coff = c * ch
            idx_vec = idx_v[pl.ds(coff, ch)]
            descs = []
            for j in range(ch):
                r = idx_vec[j]
                descs.append(
                    pltpu.async_copy(
                        table_hbm.at[pl.ds(r, 1)], rows_v.at[pl.ds(j, 1)], sem
                    )
                )
            for dsc in descs:
                dsc.wait()
            pltpu.sync_copy(rows_v, out_hbm.at[pl.ds(base + coff, ch)])
            return carry

        lax.fori_loop(0, n_chunks, chunk_body, 0)

    return sc_gather


def _pool_body(g_ref, x_ref):
    g = g_ref[...]
    ss = jnp.sum(g * g, axis=-1, keepdims=True)
    norm = jnp.sqrt(ss)
    scale = jnp.minimum(1.0, 1.0 / jnp.maximum(norm, 1e-12))
    x_ref[...] = jnp.mean(g * scale, axis=1)


def _pool(gathered):
    bt = 256
    return pl.pallas_call(
        _pool_body,
        grid=(B // bt,),
        in_specs=[pl.BlockSpec((bt, CTX, D), lambda i: (i, 0, 0))],
        out_specs=pl.BlockSpec((bt, D), lambda i: (i, 0)),
        out_shape=jax.ShapeDtypeStruct((B, D), jnp.float32),
    )(gathered)


def _matmul_body(x_ref, w_ref, b_ref, o_ref):
    xb = x_ref[...].astype(jnp.bfloat16)
    wb = w_ref[...].astype(jnp.bfloat16)
    acc = lax.dot_general(
        xb, wb, (((1,), (1,)), ((), ())),
        preferred_element_type=jnp.float32,
    )
    o_ref[...] = acc + b_ref[...]


def _matmul(x, w, b2):
    v = w.shape[0]
    vt = 4096
    return pl.pallas_call(
        _matmul_body,
        grid=(pl.cdiv(v, vt),),
        in_specs=[
            pl.BlockSpec((B, D), lambda j: (0, 0)),
            pl.BlockSpec((vt, D), lambda j: (j, 0)),
            pl.BlockSpec((1, vt), lambda j: (0, j)),
        ],
        out_specs=pl.BlockSpec((B, vt), lambda j: (0, j)),
        out_shape=jax.ShapeDtypeStruct((B, v), jnp.float32),
    )(x, w, b2)


def kernel(inputs_, emb_table, W, b):
    idx_flat = inputs_.reshape(-1)
    x = _make_sc_pool()(emb_table, idx_flat).reshape(B, D)
    return _matmul(x, W, b.reshape(1, -1))


# speedup vs baseline: 1.9078x; 1.0019x over previous
import functools

import jax
import jax.numpy as jnp
from jax import lax
from jax.experimental import pallas as pl
from jax.experimental.pallas import tpu as pltpu
from jax.experimental.pallas import tpu_sc as plsc

D = 300
B = 1024
CTX = 20
NROWS = B * CTX


_RSQRT_MAGIC = 0x5F3759DF


@functools.lru_cache(maxsize=None)
def _make_sc_pool():
    info = plsc.get_sparse_core_info()
    nw = info.num_cores * info.num_subcores
    bat_per_w = B // nw
    n_full = D // 16
    tail = D - n_full * 16
    tail_off = D - 16
    mesh = plsc.VectorSubcoreMesh(core_axis_name="c", subcore_axis_name="s")

    @functools.partial(
        pl.kernel,
        mesh=mesh,
        compiler_params=pltpu.CompilerParams(needs_layout_passes=False),
        out_type=jax.ShapeDtypeStruct((B * D,), jnp.float32),
        scratch_types=[
            pltpu.VMEM((bat_per_w * CTX,), jnp.int32),
            pltpu.VMEM((CTX, D), jnp.float32),
            pltpu.VMEM((CTX, D), jnp.float32),
            pltpu.VMEM((bat_per_w * D,), jnp.float32),
            pltpu.SemaphoreType.DMA,
            pltpu.SemaphoreType.DMA,
        ],
    )
    def sc_pool(table_hbm, idx_hbm, x_hbm, idx_v, rows_a, rows_b, xst, sem_a, sem_b):
        wid = lax.axis_index("s") * info.num_cores + lax.axis_index("c")
        pltpu.sync_copy(
            idx_hbm.at[pl.ds(wid * bat_per_w * CTX, bat_per_w * CTX)], idx_v
        )
        tail_mask = lax.iota(jnp.int32, 16) >= (16 - tail)

        def fire(bl, buf, sem):
            ioff = bl * CTX
            iv0 = idx_v[pl.ds(ioff, 16)]
            iv1 = idx_v[pl.ds(ioff + CTX - 16, 16)]
            for j in range(CTX):
                r = iv0[j] if j < 16 else iv1[j - (CTX - 16)]
                pltpu.async_copy(
                    table_hbm.at[pl.ds(r, 1)], buf.at[pl.ds(j, 1)], sem
                )

        def drain(buf, sem):
            for j in range(CTX):
                pltpu.make_async_copy(
                    table_hbm.at[pl.ds(0, 1)], buf.at[pl.ds(j, 1)], sem
                ).wait()

        def compute(bl, buf):
            xacc = [jnp.zeros((16,), jnp.float32) for _ in range(n_full + 1)]
            for j in range(CTX):
                chunks = [buf[j, pl.ds(c * 16, 16)] for c in range(n_full)]
                last = buf[j, pl.ds(tail_off, 16)]
                last = jnp.where(tail_mask, last, 0.0)
                chunks.append(last)
                ssq = jnp.zeros((16,), jnp.float32)
                for ch_ in chunks:
                    ssq = ssq + ch_ * ch_
                s = jnp.sum(ssq)
                svec = jnp.full((16,), s, jnp.float32)
                ibits = lax.bitcast_convert_type(svec, jnp.int32)
                y = lax.bitcast_convert_type(
                    _RSQRT_MAGIC - lax.shift_right_logical(ibits, 1), jnp.float32
                )
                for _ in range(3):
                    y = y * (1.5 - 0.5 * svec * y * y)
                scale = jnp.where(svec > 1.0, y, 1.0)
                for c in range(n_full + 1):
                    xacc[c] = xacc[c] + chunks[c] * scale

            inv = jnp.float32(1.0 / CTX)
            xoff = bl * D
            xst[pl.ds(xoff + tail_off, 16)] = xacc[n_full] * inv
            for c in range(n_full):
                xst[pl.ds(xoff + c * 16, 16)] = xacc[c] * inv

        fire(0, rows_a, sem_a)

        def body(k, carry):
            b0 = 2 * k
            fire(b0 + 1, rows_b, sem_b)
            drain(rows_a, sem_a)
            compute(b0, rows_a)

            @pl.when(k < bat_per_w // 2 - 1)
            def _():
                fire(b0 + 2, rows_a, sem_a)

            drain(rows_b, sem_b)
            compute(b0 + 1, rows_b)
            return carry

        lax.fori_loop(0, bat_per_w // 2, body, 0)
        pltpu.sync_copy(xst, x_hbm.at[pl.ds(wid * bat_per_w * D, bat_per_w * D)])

    return sc_pool


@functools.lru_cache(maxsize=None)
def _make_sc_gather(n_rows: int, d: int):
    info = plsc.get_sparse_core_info()
    nw = info.num_cores * info.num_subcores
    rows_per_w = n_rows // nw
    ch = 16
    n_chunks = rows_per_w // ch
    assert rows_per_w % ch == 0

    mesh = plsc.VectorSubcoreMesh(core_axis_name="c", subcore_axis_name="s")

    @functools.partial(
        pl.kernel,
        mesh=mesh,
        out_type=jax.ShapeDtypeStruct((n_rows, d), jnp.float32),
        scratch_types=[
            pltpu.VMEM((rows_per_w,), jnp.int32),
            pltpu.VMEM((ch, d), jnp.float32),
            pltpu.SemaphoreType.DMA,
        ],
    )
    def sc_gather(table_hbm, idx_hbm, out_hbm, idx_v, rows_v, sem):
        wid = lax.axis_index("s") * info.num_cores + lax.axis_index("c")
        base = wid * rows_per_w
        pltpu.sync_copy(idx_hbm.at[pl.ds(base, rows_per_w)], idx_v)

        def chunk_body(c, carry):
            coff = c * ch
            idx_vec = idx_v[pl.ds(coff, ch)]
            descs = []
            for j in range(ch):
                r = idx_vec[j]
                descs.append(
                    pltpu.async_copy(
                        table_hbm.at[pl.ds(r, 1)], rows_v.at[pl.ds(j, 1)], sem
                    )
                )
            for dsc in descs:
                dsc.wait()
            pltpu.sync_copy(rows_v, out_hbm.at[pl.ds(base + coff, ch)])
            return carry

        lax.fori_loop(0, n_chunks, chunk_body, 0)

    return sc_gather


def _pool_body(g_ref, x_ref):
    g = g_ref[...]
    ss = jnp.sum(g * g, axis=-1, keepdims=True)
    norm = jnp.sqrt(ss)
    scale = jnp.minimum(1.0, 1.0 / jnp.maximum(norm, 1e-12))
    x_ref[...] = jnp.mean(g * scale, axis=1)


def _pool(gathered):
    bt = 256
    return pl.pallas_call(
        _pool_body,
        grid=(B // bt,),
        in_specs=[pl.BlockSpec((bt, CTX, D), lambda i: (i, 0, 0))],
        out_specs=pl.BlockSpec((bt, D), lambda i: (i, 0)),
        out_shape=jax.ShapeDtypeStruct((B, D), jnp.float32),
    )(gathered)


def _matmul_body(x_ref, w_ref, b_ref, o_ref):
    xb = x_ref[...].astype(jnp.bfloat16)
    wb = w_ref[...].astype(jnp.bfloat16)
    acc = lax.dot_general(
        xb, wb, (((1,), (1,)), ((), ())),
        preferred_element_type=jnp.float32,
    )
    o_ref[...] = acc + b_ref[...]


def _matmul(x, w, b2):
    v = w.shape[0]
    vt = 4096
    return pl.pallas_call(
        _matmul_body,
        grid=(pl.cdiv(v, vt),),
        in_specs=[
            pl.BlockSpec((B, D), lambda j: (0, 0)),
            pl.BlockSpec((vt, D), lambda j: (j, 0)),
            pl.BlockSpec((1, vt), lambda j: (0, j)),
        ],
        out_specs=pl.BlockSpec((B, vt), lambda j: (0, j)),
        out_shape=jax.ShapeDtypeStruct((B, v), jnp.float32),
    )(x, w, b2)


def kernel(inputs_, emb_table, W, b):
    idx_flat = inputs_.reshape(-1)
    x = _make_sc_pool()(emb_table, idx_flat).reshape(B, D)
    return _matmul(x, W, b.reshape(1, -1))


# speedup vs baseline: 1.9152x; 1.0039x over previous
import functools

import jax
import jax.numpy as jnp
from jax import lax
from jax.experimental import pallas as pl
from jax.experimental.pallas import tpu as pltpu
from jax.experimental.pallas import tpu_sc as plsc

D = 300
B = 1024
CTX = 20
NROWS = B * CTX


_RSQRT_MAGIC = 0x5F3759DF


@functools.lru_cache(maxsize=None)
def _make_sc_pool():
    info = plsc.get_sparse_core_info()
    nw = info.num_cores * info.num_subcores
    bat_per_w = B // nw
    n_full = D // 16
    tail = D - n_full * 16
    tail_off = D - 16
    mesh = plsc.VectorSubcoreMesh(core_axis_name="c", subcore_axis_name="s")

    @functools.partial(
        pl.kernel,
        mesh=mesh,
        compiler_params=pltpu.CompilerParams(needs_layout_passes=False),
        out_type=jax.ShapeDtypeStruct((B, D), jnp.float32),
        scratch_types=[
            pltpu.VMEM((bat_per_w * CTX,), jnp.int32),
            pltpu.VMEM((CTX, D), jnp.float32),
            pltpu.VMEM((CTX, D), jnp.float32),
            pltpu.VMEM((bat_per_w, D), jnp.float32),
            pltpu.SemaphoreType.DMA,
            pltpu.SemaphoreType.DMA,
        ],
    )
    def sc_pool(table_hbm, idx_hbm, x_hbm, idx_v, rows_a, rows_b, xst, sem_a, sem_b):
        wid = lax.axis_index("s") * info.num_cores + lax.axis_index("c")
        pltpu.sync_copy(
            idx_hbm.at[pl.ds(wid * bat_per_w * CTX, bat_per_w * CTX)], idx_v
        )
        tail_mask = lax.iota(jnp.int32, 16) >= (16 - tail)

        def fire(bl, buf, sem):
            ioff = bl * CTX
            iv0 = idx_v[pl.ds(ioff, 16)]
            iv1 = idx_v[pl.ds(ioff + CTX - 16, 16)]
            for j in range(CTX):
                r = iv0[j] if j < 16 else iv1[j - (CTX - 16)]
                pltpu.async_copy(
                    table_hbm.at[pl.ds(r, 1)], buf.at[pl.ds(j, 1)], sem
                )

        def drain(buf, sem):
            for j in range(CTX):
                pltpu.make_async_copy(
                    table_hbm.at[pl.ds(0, 1)], buf.at[pl.ds(j, 1)], sem
                ).wait()

        def compute(bl, buf):
            xacc = [jnp.zeros((16,), jnp.float32) for _ in range(n_full + 1)]
            for j in range(CTX):
                chunks = [buf[j, pl.ds(c * 16, 16)] for c in range(n_full)]
                last = buf[j, pl.ds(tail_off, 16)]
                last = jnp.where(tail_mask, last, 0.0)
                chunks.append(last)
                ssq = jnp.zeros((16,), jnp.float32)
                for ch_ in chunks:
                    ssq = ssq + ch_ * ch_
                s = jnp.sum(ssq)
                svec = jnp.full((16,), s, jnp.float32)
                ibits = lax.bitcast_convert_type(svec, jnp.int32)
                y = lax.bitcast_convert_type(
                    _RSQRT_MAGIC - lax.shift_right_logical(ibits, 1), jnp.float32
                )
                for _ in range(3):
                    y = y * (1.5 - 0.5 * svec * y * y)
                scale = jnp.where(svec > 1.0, y, 1.0)
                for c in range(n_full + 1):
                    xacc[c] = xacc[c] + chunks[c] * scale

            inv = jnp.float32(1.0 / CTX)
            xst[bl, pl.ds(tail_off, 16)] = xacc[n_full] * inv
            for c in range(n_full):
                xst[bl, pl.ds(c * 16, 16)] = xacc[c] * inv

        fire(0, rows_a, sem_a)

        def body(k, carry):
            b0 = 2 * k
            fire(b0 + 1, rows_b, sem_b)
            drain(rows_a, sem_a)
            compute(b0, rows_a)

            @pl.when(k < bat_per_w // 2 - 1)
            def _():
                fire(b0 + 2, rows_a, sem_a)

            drain(rows_b, sem_b)
            compute(b0 + 1, rows_b)
            return carry

        lax.fori_loop(0, bat_per_w // 2, body, 0)
        pltpu.sync_copy(xst, x_hbm.at[pl.ds(wid * bat_per_w, bat_per_w)])

    return sc_pool


@functools.lru_cache(maxsize=None)
def _make_sc_gather(n_rows: int, d: int):
    info = plsc.get_sparse_core_info()
    nw = info.num_cores * info.num_subcores
    rows_per_w = n_rows // nw
    ch = 16
    n_chunks = rows_per_w // ch
    assert rows_per_w % ch == 0

    mesh = plsc.VectorSubcoreMesh(core_axis_name="c", subcore_axis_name="s")

    @functools.partial(
        pl.kernel,
        mesh=mesh,
        out_type=jax.ShapeDtypeStruct((n_rows, d), jnp.float32),
        scratch_types=[
            pltpu.VMEM((rows_per_w,), jnp.int32),
            pltpu.VMEM((ch, d), jnp.float32),
            pltpu.SemaphoreType.DMA,
        ],
    )
    def sc_gather(table_hbm, idx_hbm, out_hbm, idx_v, rows_v, sem):
        wid = lax.axis_index("s") * info.num_cores + lax.axis_index("c")
        base = wid * rows_per_w
        pltpu.sync_copy(idx_hbm.at[pl.ds(base, rows_per_w)], idx_v)

        def chunk_body(c, carry):
            coff = c * ch
            idx_vec = idx_v[pl.ds(coff, ch)]
            descs = []
            for j in range(ch):
                r = idx_vec[j]
                descs.append(
                    pltpu.async_copy(
                        table_hbm.at[pl.ds(r, 1)], rows_v.at[pl.ds(j, 1)], sem
                    )
                )
            for dsc in descs:
                dsc.wait()
            pltpu.sync_copy(rows_v, out_hbm.at[pl.ds(base + coff, ch)])
            return carry

        lax.fori_loop(0, n_chunks, chunk_body, 0)

    return sc_gather


def _pool_body(g_ref, x_ref):
    g = g_ref[...]
    ss = jnp.sum(g * g, axis=-1, keepdims=True)
    norm = jnp.sqrt(ss)
    scale = jnp.minimum(1.0, 1.0 / jnp.maximum(norm, 1e-12))
    x_ref[...] = jnp.mean(g * scale, axis=1)


def _pool(gathered):
    bt = 256
    return pl.pallas_call(
        _pool_body,
        grid=(B // bt,),
        in_specs=[pl.BlockSpec((bt, CTX, D), lambda i: (i, 0, 0))],
        out_specs=pl.BlockSpec((bt, D), lambda i: (i, 0)),
        out_shape=jax.ShapeDtypeStruct((B, D), jnp.float32),
    )(gathered)


def _matmul_body(x_ref, w_ref, b_ref, o_ref):
    xb = x_ref[...].astype(jnp.bfloat16)
    wb = w_ref[...].astype(jnp.bfloat16)
    acc = lax.dot_general(
        xb, wb, (((1,), (1,)), ((), ())),
        preferred_element_type=jnp.float32,
    )
    o_ref[...] = acc + b_ref[...]


def _matmul(x, w, b2):
    v = w.shape[0]
    vt = 4096
    return pl.pallas_call(
        _matmul_body,
        grid=(pl.cdiv(v, vt),),
        in_specs=[
            pl.BlockSpec((B, D), lambda j: (0, 0)),
            pl.BlockSpec((vt, D), lambda j: (j, 0)),
            pl.BlockSpec((1, vt), lambda j: (0, j)),
        ],
        out_specs=pl.BlockSpec((B, vt), lambda j: (0, j)),
        out_shape=jax.ShapeDtypeStruct((B, v), jnp.float32),
    )(x, w, b2)


def kernel(inputs_, emb_table, W, b):
    idx_flat = inputs_.reshape(-1)
    x = _make_sc_pool()(emb_table, idx_flat)
    return _matmul(x, W, b.reshape(1, -1))
